# baseline (speedup 1.0000x reference)
import jax
import jax.numpy as jnp
from jax.experimental import pallas as pl

N0 = 100000
N1 = 25000
N2 = 6250


def _id_pallas(x):
    def body(x_ref, o_ref):
        o_ref[...] = x_ref[...]

    blk = 1000
    return pl.pallas_call(
        body,
        out_shape=jax.ShapeDtypeStruct(x.shape, x.dtype),
        grid=(x.shape[0] // blk,),
        in_specs=[pl.BlockSpec((blk, x.shape[1]), lambda i: (i, 0))],
        out_specs=pl.BlockSpec((blk, x.shape[1]), lambda i: (i, 0)),
    )(x)


def _spconv(x, src, dst, Ws, Wn, b, n):
    msg = jnp.take(x @ Wn, src, axis=0)
    agg = jax.ops.segment_sum(msg, dst, num_segments=n)
    return x @ Ws + agg + b


def _bn(x, g, be):
    m = jnp.mean(x, axis=0)
    v = jnp.var(x, axis=0)
    return (x - m) * jax.lax.rsqrt(v + 1e-5) * g + be


def _res(x, src, dst, p, nm, n):
    h = jax.nn.relu(_bn(_spconv(x, src, dst, p[nm + "_c1_Ws"], p[nm + "_c1_Wn"], p[nm + "_c1_b"], n), p[nm + "_bn1_g"], p[nm + "_bn1_be"]))
    h = _bn(_spconv(h, src, dst, p[nm + "_c2_Ws"], p[nm + "_c2_Wn"], p[nm + "_c2_b"], n), p[nm + "_bn2_g"], p[nm + "_bn2_be"])
    return jax.nn.relu(h + x)


def kernel(x, params, edge_index0, edge_index1, edge_index2, pool_idx1, pool_idx2):
    p = params
    s0, d0 = edge_index0[0], edge_index0[1]
    s1, d1 = edge_index1[0], edge_index1[1]
    s2, d2 = edge_index2[0], edge_index2[1]
    x = _id_pallas(x)
    x0 = jax.nn.relu(_bn(_spconv(x, s0, d0, p["stem_c_Ws"], p["stem_c_Wn"], p["stem_c_b"], N0), p["stem_bn_g"], p["stem_bn_be"]))
    x1 = _res(x0, s0, d0, p, "enc1", N0)
    c1 = jax.ops.segment_sum(x1 @ p["down1_W"], pool_idx1, num_segments=N1) + p["down1_b"]
    x2 = _res(c1, s1, d1, p, "enc2", N1)
    c2 = jax.ops.segment_sum(x2 @ p["down2_W"], pool_idx2, num_segments=N2) + p["down2_b"]
    x3 = _res(c2, s2, d2, p, "bott", N2)
    u1 = jnp.take(x3 @ p["up1_W"], pool_idx2, axis=0) + p["up1_b"]
    y1 = _res(u1, s1, d1, p, "dec1", N1)
    u2 = jnp.take(y1 @ p["up2_W"], pool_idx1, axis=0) + p["up2_b"]
    y0 = _res(u2, s0, d0, p, "dec2", N0)
    return y0 @ p["head_W"] + p["head_b"]


# speedup vs baseline: 3.9565x; 3.9565x over previous
import functools

import jax
import jax.numpy as jnp
from jax import lax
from jax.experimental import pallas as pl
from jax.experimental.pallas import tpu as pltpu
from jax.experimental.pallas import tpu_sc as plsc

N0 = 100000
N1 = 25000
N2 = 6250

NC = 2
NS = 16
K = 128


def _rup(x, m):
    return (x + m - 1) // m * m


def _mesh():
    return plsc.VectorSubcoreMesh(core_axis_name="c", subcore_axis_name="s")


_SC_PARAMS = pltpu.CompilerParams(use_tc_tiling_on_sc=False)


def _zero_rows(rows, h):
    zv = jnp.zeros((16,), jnp.float32)

    def zb(i, carry):
        for j in range(h // 16):
            rows[i, pl.ds(j * 16, 16)] = zv
        return carry

    lax.fori_loop(0, K, zb, 0)


@functools.partial(jax.jit, static_argnames=("n_out", "edge_split"))
def _segsum(xA, xB, srcp, dstp, n_out, edge_split=False):
    e_pad = srcp.shape[0]
    h = xA.shape[1]
    n_pad = _rup(n_out + K, NS * K)
    span_e = e_pad // (NC * NS if edge_split else NS)
    n_chunks = span_e // K
    zspan = n_pad // NS
    nz = zspan // K

    @functools.partial(
        pl.kernel,
        out_type=jax.ShapeDtypeStruct((NC, n_pad, h), jnp.float32),
        mesh=_mesh(),
        scratch_types=[
            pltpu.VMEM((K,), jnp.int32),
            pltpu.VMEM((K,), jnp.int32),
            pltpu.VMEM((K, h), jnp.float32),
            pltpu.VMEM_SHARED((n_pad, h), jnp.float32),
            pltpu.SemaphoreType.DMA,
        ],
        compiler_params=_SC_PARAMS,
    )
    def k(xA_h, xB_h, src_h, dst_h, out_h, idx_s, idx_d, rows, acc, sem):
        c = lax.axis_index("c")
        s = lax.axis_index("s")
        base = s * zspan

        _zero_rows(rows, h)

        def zc(i, carry):
            pltpu.sync_copy(rows, acc.at[pl.ds(base + i * K, K)])
            return carry

        lax.fori_loop(0, nz, zc, 0)
        plsc.subcore_barrier()

        if edge_split:
            eb = (c * NS + s) * span_e
        else:
            eb = s * span_e

        def ebody(g, carry):
            off = eb + g * K
            pltpu.sync_copy(src_h.at[pl.ds(off, K)], idx_s)
            pltpu.sync_copy(dst_h.at[pl.ds(off, K)], idx_d)
            if edge_split:
                pltpu.async_copy(xA_h.at[idx_s], rows, sem).wait()
            else:
                @pl.when(c == 0)
                def _():
                    pltpu.async_copy(xA_h.at[idx_s], rows, sem).wait()

                @pl.when(c == 1)
                def _():
                    pltpu.async_copy(xB_h.at[idx_s], rows, sem).wait()

            pltpu.sync_copy(rows, acc.at[idx_d], add=True)
            return carry

        lax.fori_loop(0, n_chunks, ebody, 0)
        plsc.subcore_barrier()

        def dr(i, carry):
            sl = pl.ds(base + i * K, K)
            pltpu.sync_copy(acc.at[sl], out_h.at[c, sl])
            return carry

        lax.fori_loop(0, nz, dr, 0)

    out = k(xA, xB, srcp, dstp)
    if edge_split:
        return out[0, :n_out] + out[1, :n_out]
    return jnp.concatenate([out[0, :n_out], out[1, :n_out]], axis=1)


@functools.partial(jax.jit, static_argnames=("n_out",))
def _take_rows(tA, tB, idxp, n_out):
    h = tA.shape[1]
    n_outp = idxp.shape[0]
    span = n_outp // NS
    n_chunks = span // K

    @functools.partial(
        pl.kernel,
        out_type=jax.ShapeDtypeStruct((NC, n_outp, h), jnp.float32),
        mesh=_mesh(),
        scratch_types=[
            pltpu.VMEM((K,), jnp.int32),
            pltpu.VMEM((K, h), jnp.float32),
            pltpu.SemaphoreType.DMA,
        ],
        compiler_params=_SC_PARAMS,
    )
    def k(tA_h, tB_h, idx_h, out_h, idx_v, rows, sem):
        c = lax.axis_index("c")
        s = lax.axis_index("s")

        def body(g, carry):
            off = s * span + g * K
            pltpu.sync_copy(idx_h.at[pl.ds(off, K)], idx_v)

            @pl.when(c == 0)
            def _():
                pltpu.async_copy(tA_h.at[idx_v], rows, sem).wait()

            @pl.when(c == 1)
            def _():
                pltpu.async_copy(tB_h.at[idx_v], rows, sem).wait()

            pltpu.sync_copy(rows, out_h.at[c, pl.ds(off, K)])
            return carry

        lax.fori_loop(0, n_chunks, body, 0)

    out = k(tA, tB, idxp)
    return jnp.concatenate([out[0, :n_out], out[1, :n_out]], axis=1)


def _pad_edges(src, dst, n_out, nworkers):
    unit = nworkers * K
    e = src.shape[0]
    e_pad = _rup(e, unit)
    pad = e_pad - e
    srcp = jnp.concatenate([src, jnp.zeros((pad,), jnp.int32)])
    dstp = jnp.concatenate([dst, jnp.full((pad,), n_out, jnp.int32)])
    return srcp, dstp


def _pad_idx(idx):
    n = idx.shape[0]
    n_pad = _rup(n, NS * K)
    return jnp.concatenate([idx, jnp.zeros((n_pad - n,), jnp.int32)])


def _halves(x):
    hh = x.shape[1] // 2
    return x[:, :hh], x[:, hh:]


def _mm(a, b):
    return jnp.dot(a, b, precision=lax.Precision.HIGHEST)


def _bn(x, g, be):
    m = jnp.mean(x, axis=0)
    v = jnp.var(x, axis=0)
    return (x - m) * lax.rsqrt(v + 1e-5) * g + be


def _conv(x, agg, Ws, Wn, b):
    return _mm(x, Ws) + _mm(agg, Wn) + b


def _res(x, srcp, dstp, p, nm, n):
    xA, xB = _halves(x)
    g1 = _segsum(xA, xB, srcp, dstp, n)
    h = jax.nn.relu(_bn(_conv(x, g1, p[nm + "_c1_Ws"], p[nm + "_c1_Wn"], p[nm + "_c1_b"]),
                        p[nm + "_bn1_g"], p[nm + "_bn1_be"]))
    hA, hB = _halves(h)
    g2 = _segsum(hA, hB, srcp, dstp, n)
    h = _bn(_conv(h, g2, p[nm + "_c2_Ws"], p[nm + "_c2_Wn"], p[nm + "_c2_b"]),
            p[nm + "_bn2_g"], p[nm + "_bn2_be"])
    return jax.nn.relu(h + x)


def kernel(x, params, edge_index0, edge_index1, edge_index2, pool_idx1, pool_idx2):
    p = params
    s0p, d0p = _pad_edges(edge_index0[0], edge_index0[1], N0, NS)
    s1p, d1p = _pad_edges(edge_index1[0], edge_index1[1], N1, NS)
    s2p, d2p = _pad_edges(edge_index2[0], edge_index2[1], N2, NS)
    s0ep, d0ep = _pad_edges(edge_index0[0], edge_index0[1], N0, NC * NS)
    ar0 = jnp.arange(N0, dtype=jnp.int32)
    ar1 = jnp.arange(N1, dtype=jnp.int32)
    pool1s, pool1d = _pad_edges(ar0, pool_idx1.astype(jnp.int32), N1, NS)
    pool2s, pool2d = _pad_edges(ar1, pool_idx2.astype(jnp.int32), N2, NS)

    x16 = jnp.pad(x, ((0, 0), (0, 9)))
    agg_s = _segsum(x16, x16, s0ep, d0ep, N0, edge_split=True)
    Wn16 = jnp.pad(p["stem_c_Wn"], ((0, 9), (0, 0)))
    z = _mm(x, p["stem_c_Ws"]) + _mm(agg_s, Wn16) + p["stem_c_b"]
    x0 = jax.nn.relu(_bn(z, p["stem_bn_g"], p["stem_bn_be"]))

    x1 = _res(x0, s0p, d0p, p, "enc1", N0)

    x1A, x1B = _halves(x1)
    pool1 = _segsum(x1A, x1B, pool1s, pool1d, N1)
    c1 = _mm(pool1, p["down1_W"]) + p["down1_b"]

    x2 = _res(c1, s1p, d1p, p, "enc2", N1)

    x2A, x2B = _halves(x2)
    pool2 = _segsum(x2A, x2B, pool2s, pool2d, N2)
    c2 = _mm(pool2, p["down2_W"]) + p["down2_b"]

    x3 = _res(c2, s2p, d2p, p, "bott", N2)

    t1 = _mm(x3, p["up1_W"]) + p["up1_b"]
    t1A, t1B = _halves(t1)
    u1 = _take_rows(t1A, t1B, _pad_idx(pool_idx2.astype(jnp.int32)), N1)

    y1 = _res(u1, s1p, d1p, p, "dec1", N1)

    t2 = _mm(y1, p["up2_W"]) + p["up2_b"]
    t2A, t2B = _halves(t2)
    u2 = _take_rows(t2A, t2B, _pad_idx(pool_idx1.astype(jnp.int32)), N0)

    y0 = _res(u2, s0p, d0p, p, "dec2", N0)
    return _mm(y0, p["head_W"]) + p["head_b"]


# speedup vs baseline: 6.8618x; 1.7343x over previous
import functools

import jax
import jax.numpy as jnp
from jax import lax
from jax.experimental import pallas as pl
from jax.experimental.pallas import tpu as pltpu
from jax.experimental.pallas import tpu_sc as plsc

N0 = 100000
N1 = 25000
N2 = 6250

NC = 2
NS = 16
GL = 512
NG = 2


def _rup(x, m):
    return (x + m - 1) // m * m


def _mesh():
    return plsc.VectorSubcoreMesh(core_axis_name="c", subcore_axis_name="s")


_SC_PARAMS = pltpu.CompilerParams(use_tc_tiling_on_sc=False)


@functools.partial(jax.jit, static_argnames=("n_out", "edge_split"))
def _segsum(xA, xB, srcp, dstp, n_out, edge_split=False):
    e_pad = srcp.shape[0]
    h = xA.shape[1]
    n_pad = _rup(n_out + 8, NS * GL)
    ept = e_pad // (NC * NS if edge_split else NS)
    nstep = ept // (NG * GL)
    total_groups = nstep * NG
    zspan = n_pad // NS
    nz = zspan // GL

    @functools.partial(
        pl.kernel,
        out_type=jax.ShapeDtypeStruct((NC, n_pad, h), jnp.float32),
        mesh=_mesh(),
        scratch_types=[
            [pltpu.VMEM((GL,), jnp.int32)] * NG,
            [pltpu.VMEM((GL,), jnp.int32)] * NG,
            [pltpu.VMEM((GL, h), jnp.float32)] * NG,
            pltpu.VMEM_SHARED((n_pad, h), jnp.float32),
            pltpu.SemaphoreType.DMA,
            pltpu.SemaphoreType.DMA,
            pltpu.SemaphoreType.DMA,
        ],
        compiler_params=_SC_PARAMS,
    )
    def k(xA_h, xB_h, src_h, dst_h, out_h, sbufs, dbufs, rowss, acc, sem_i, sem_g, sem_s):
        c = lax.axis_index("c")
        s = lax.axis_index("s")
        base = s * zspan
        zv = jnp.zeros((16,), jnp.float32)

        def zb(i, carry):
            for j in range(h // 16):
                rowss[0][i, pl.ds(j * 16, 16)] = zv
            return carry

        lax.fori_loop(0, GL, zb, 0)

        def zc(i, carry):
            pltpu.sync_copy(rowss[0], acc.at[pl.ds(base + i * GL, GL)])
            return carry

        lax.fori_loop(0, nz, zc, 0)
        plsc.subcore_barrier()

        if edge_split:
            ebase = (c * NS + s) * ept
        else:
            ebase = s * ept

        def load_group(m, grp):
            off = ebase + m * GL
            pltpu.async_copy(src_h.at[pl.ds(off, GL)], sbufs[grp], sem_i)
            pltpu.async_copy(dst_h.at[pl.ds(off, GL)], dbufs[grp], sem_i)

        def wait_idx():
            pltpu.make_async_copy(src_h.at[pl.ds(0, GL)], sbufs[0], sem_i).wait()
            pltpu.make_async_copy(dst_h.at[pl.ds(0, GL)], dbufs[0], sem_i).wait()

        def fire_gather(grp):
            if edge_split:
                pltpu.async_copy(xA_h.at[sbufs[grp]], rowss[grp], sem_g)
            else:
                @pl.when(c == 0)
                def _():
                    pltpu.async_copy(xA_h.at[sbufs[grp]], rowss[grp], sem_g)

                @pl.when(c == 1)
                def _():
                    pltpu.async_copy(xB_h.at[sbufs[grp]], rowss[grp], sem_g)

        def wait_gather(grp):
            pltpu.make_async_copy(xA_h.at[sbufs[grp]], rowss[grp], sem_g).wait()

        def fire_scatter(grp):
            pltpu.async_copy(rowss[grp], acc.at[dbufs[grp]], sem_s, add=True)

        def wait_scatter(grp):
            pltpu.make_async_copy(rowss[grp], acc.at[dbufs[grp]], sem_s).wait()

        load_group(0, 0)
        wait_idx()
        fire_gather(0)

        def body(m, carry):
            for grp in range(NG):
                g_idx = m * NG + grp
                nxt = (grp + 1) % NG

                @pl.when(g_idx + 1 < total_groups)
                def _():
                    load_group(g_idx + 1, nxt)
                    wait_idx()
                    fire_gather(nxt)

                wait_gather(grp)
                fire_scatter(grp)
                wait_scatter(grp)
            return carry

        lax.fori_loop(0, nstep, body, 0)
        plsc.subcore_barrier()

        def dr(i, carry):
            sl = pl.ds(base + i * GL, GL)
            pltpu.sync_copy(acc.at[sl], out_h.at[c, sl])
            return carry

        lax.fori_loop(0, nz, dr, 0)

    out = k(xA, xB, srcp, dstp)
    if edge_split:
        return out[0, :n_out] + out[1, :n_out]
    return jnp.concatenate([out[0, :n_out], out[1, :n_out]], axis=1)


@functools.partial(jax.jit, static_argnames=("n_out",))
def _take_rows(tA, tB, idxp, n_out):
    h = tA.shape[1]
    n_outp = idxp.shape[0]
    span = n_outp // NS
    nstep = span // (NG * GL)
    total_groups = nstep * NG

    @functools.partial(
        pl.kernel,
        out_type=jax.ShapeDtypeStruct((NC, n_outp, h), jnp.float32),
        mesh=_mesh(),
        scratch_types=[
            [pltpu.VMEM((GL,), jnp.int32)] * NG,
            [pltpu.VMEM((GL, h), jnp.float32)] * NG,
            pltpu.SemaphoreType.DMA,
            pltpu.SemaphoreType.DMA,
            pltpu.SemaphoreType.DMA,
        ],
        compiler_params=_SC_PARAMS,
    )
    def k(tA_h, tB_h, idx_h, out_h, ibufs, rowss, sem_i, sem_g, sem_o):
        c = lax.axis_index("c")
        s = lax.axis_index("s")
        ebase = s * span

        def load_group(m, grp):
            pltpu.async_copy(idx_h.at[pl.ds(ebase + m * GL, GL)], ibufs[grp], sem_i)

        def wait_idx():
            pltpu.make_async_copy(idx_h.at[pl.ds(0, GL)], ibufs[0], sem_i).wait()

        def fire_gather(grp):
            @pl.when(c == 0)
            def _():
                pltpu.async_copy(tA_h.at[ibufs[grp]], rowss[grp], sem_g)

            @pl.when(c == 1)
            def _():
                pltpu.async_copy(tB_h.at[ibufs[grp]], rowss[grp], sem_g)

        def wait_gather(grp):
            pltpu.make_async_copy(tA_h.at[ibufs[grp]], rowss[grp], sem_g).wait()

        def fire_out(m, grp):
            pltpu.async_copy(rowss[grp], out_h.at[c, pl.ds(ebase + m * GL, GL)], sem_o)

        def wait_out(m, grp):
            pltpu.make_async_copy(rowss[grp], out_h.at[c, pl.ds(ebase + m * GL, GL)], sem_o).wait()

        load_group(0, 0)
        wait_idx()
        fire_gather(0)

        def body(m, carry):
            for grp in range(NG):
                g_idx = m * NG + grp
                nxt = (grp + 1) % NG

                @pl.when(g_idx + 1 < total_groups)
                def _():
                    load_group(g_idx + 1, nxt)
                    wait_idx()
                    fire_gather(nxt)

                wait_gather(grp)
                fire_out(g_idx, grp)
                wait_out(g_idx, grp)
            return carry

        lax.fori_loop(0, nstep, body, 0)

    out = k(tA, tB, idxp)
    return jnp.concatenate([out[0, :n_out], out[1, :n_out]], axis=1)


def _pad_edges(src, dst, n_out, nworkers):
    unit = nworkers * NG * GL
    e = src.shape[0]
    e_pad = _rup(e, unit)
    pad = e_pad - e
    srcp = jnp.concatenate([src, jnp.zeros((pad,), jnp.int32)])
    dstp = jnp.concatenate([dst, jnp.full((pad,), n_out, jnp.int32)])
    return srcp, dstp


def _pad_idx(idx):
    n = idx.shape[0]
    n_pad = _rup(n, NS * NG * GL)
    return jnp.concatenate([idx, jnp.zeros((n_pad - n,), jnp.int32)])


def _halves(x):
    hh = x.shape[1] // 2
    return x[:, :hh], x[:, hh:]


def _mm(a, b):
    return jnp.dot(a, b, precision=lax.Precision.HIGHEST)


def _bn(x, g, be):
    m = jnp.mean(x, axis=0)
    v = jnp.var(x, axis=0)
    return (x - m) * lax.rsqrt(v + 1e-5) * g + be


def _conv(x, agg, Ws, Wn, b):
    return _mm(x, Ws) + _mm(agg, Wn) + b


def _res(x, srcp, dstp, p, nm, n):
    xA, xB = _halves(x)
    g1 = _segsum(xA, xB, srcp, dstp, n)
    h = jax.nn.relu(_bn(_conv(x, g1, p[nm + "_c1_Ws"], p[nm + "_c1_Wn"], p[nm + "_c1_b"]),
                        p[nm + "_bn1_g"], p[nm + "_bn1_be"]))
    hA, hB = _halves(h)
    g2 = _segsum(hA, hB, srcp, dstp, n)
    h = _bn(_conv(h, g2, p[nm + "_c2_Ws"], p[nm + "_c2_Wn"], p[nm + "_c2_b"]),
            p[nm + "_bn2_g"], p[nm + "_bn2_be"])
    return jax.nn.relu(h + x)


def kernel(x, params, edge_index0, edge_index1, edge_index2, pool_idx1, pool_idx2):
    p = params
    s0p, d0p = _pad_edges(edge_index0[0], edge_index0[1], N0, NS)
    s1p, d1p = _pad_edges(edge_index1[0], edge_index1[1], N1, NS)
    s2p, d2p = _pad_edges(edge_index2[0], edge_index2[1], N2, NS)
    s0ep, d0ep = _pad_edges(edge_index0[0], edge_index0[1], N0, NC * NS)
    ar0 = jnp.arange(N0, dtype=jnp.int32)
    ar1 = jnp.arange(N1, dtype=jnp.int32)
    pool1s, pool1d = _pad_edges(ar0, pool_idx1.astype(jnp.int32), N1, NS)
    pool2s, pool2d = _pad_edges(ar1, pool_idx2.astype(jnp.int32), N2, NS)

    x16 = jnp.pad(x, ((0, 0), (0, 9)))
    agg_s = _segsum(x16, x16, s0ep, d0ep, N0, edge_split=True)
    Wn16 = jnp.pad(p["stem_c_Wn"], ((0, 9), (0, 0)))
    z = _mm(x, p["stem_c_Ws"]) + _mm(agg_s, Wn16) + p["stem_c_b"]
    x0 = jax.nn.relu(_bn(z, p["stem_bn_g"], p["stem_bn_be"]))

    x1 = _res(x0, s0p, d0p, p, "enc1", N0)

    x1A, x1B = _halves(x1)
    pool1 = _segsum(x1A, x1B, pool1s, pool1d, N1)
    c1 = _mm(pool1, p["down1_W"]) + p["down1_b"]

    x2 = _res(c1, s1p, d1p, p, "enc2", N1)

    x2A, x2B = _halves(x2)
    pool2 = _segsum(x2A, x2B, pool2s, pool2d, N2)
    c2 = _mm(pool2, p["down2_W"]) + p["down2_b"]

    x3 = _res(c2, s2p, d2p, p, "bott", N2)

    t1 = _mm(x3, p["up1_W"]) + p["up1_b"]
    t1A, t1B = _halves(t1)
    u1 = _take_rows(t1A, t1B, _pad_idx(pool_idx2.astype(jnp.int32)), N1)

    y1 = _res(u1, s1p, d1p, p, "dec1", N1)

    t2 = _mm(y1, p["up2_W"]) + p["up2_b"]
    t2A, t2B = _halves(t2)
    u2 = _take_rows(t2A, t2B, _pad_idx(pool_idx1.astype(jnp.int32)), N0)

    y0 = _res(u2, s0p, d0p, p, "dec2", N0)
    return _mm(y0, p["head_W"]) + p["head_b"]
